# baseline (speedup 1.0000x reference)
import math

import jax
import jax.numpy as jnp
from jax import lax
from jax.experimental import pallas as pl
from jax.experimental.pallas import tpu as pltpu
from jax.experimental.pallas import tpu_sc as plsc

_V = 1000000
_EMB = 64
_PAD = 128
_B = 4096
_L = 200
_SCALE = math.sqrt(_EMB)

_NC = 2
_NS = 16
_NW = _NC * _NS

_ROWS_W = _B // _NW
_TOK_W = _ROWS_W * _L
_SPLIT = (104, 96)
_NCHUNK = _ROWS_W
_NBUF = 4
_NOUTER = _NCHUNK // _NBUF


def _gather_body(tokens_hbm, scr_hbm, out_hbm, idx_v, rows, rows_out, *sems):
    sem_g = sems[:_NBUF]
    sem_o = sems[_NBUF:]
    wid = lax.axis_index("s") * _NC + lax.axis_index("c")

    def chunk_geom(b):
        size = _SPLIT[b % 2]
        off = 0 if b % 2 == 0 else _SPLIT[0]
        return size, off

    def idx_chunk(c, b):
        size, off = chunk_geom(b)
        return idx_v.at[pl.ds((c // 2) * _L + off, size)]

    def in_buf(b):
        return rows.at[b, pl.ds(0, chunk_geom(b)[0])]

    def emb_buf(b):
        return rows_out.at[b, pl.ds(0, chunk_geom(b)[0])]

    def run_half(h):
        row0 = wid * _ROWS_W + h * (_ROWS_W // 2)
        pltpu.sync_copy(
            tokens_hbm.at[pl.ds(wid * _TOK_W + h * (_TOK_W // 2), _TOK_W // 2)],
            idx_v,
        )

        def out_chunk(c, b):
            size, off = chunk_geom(b)
            return out_hbm.at[row0 + c // 2, pl.ds(off, size)]

        for b in range(_NBUF):
            pltpu.make_async_copy(
                scr_hbm.at[idx_chunk(b, b)], in_buf(b), sem_g[b]
            ).start()

        def round_body(g, carry):
            for b in range(_NBUF):
                c = g * _NBUF + b
                size, _ = chunk_geom(b)
                pltpu.make_async_copy(
                    scr_hbm.at[idx_chunk(c, b)], in_buf(b), sem_g[b]
                ).wait()

                @pl.when(g > 0)
                def _wait_out():
                    pltpu.make_async_copy(
                        emb_buf(b), out_chunk(c, b), sem_o[b]
                    ).wait()

                def copy_row(r, acc):
                    for j in range(_EMB // 16):
                        rows_out[b, r, pl.ds(j * 16, 16)] = (
                            rows[b, r, pl.ds(j * 16, 16)] * _SCALE
                        )
                    return acc

                lax.fori_loop(0, size, copy_row, 0, unroll=8)

                @pl.when(g < _NOUTER - 1)
                def _next_gather():
                    pltpu.make_async_copy(
                        scr_hbm.at[idx_chunk(c + _NBUF, b)], in_buf(b), sem_g[b]
                    ).start()

                pltpu.make_async_copy(emb_buf(b), out_chunk(c, b), sem_o[b]).start()

            return carry

        lax.fori_loop(0, _NOUTER, round_body, 0)

        for b in range(_NBUF):
            size, off = chunk_geom(b)
            pltpu.make_async_copy(
                emb_buf(b), out_hbm.at[row0, pl.ds(off, size)], sem_o[b]
            ).wait()

    run_half(0)
    run_half(1)


@jax.jit
def _embed(tokens_flat, tablep):
    mesh = plsc.VectorSubcoreMesh(core_axis_name="c", subcore_axis_name="s")
    return pl.kernel(
        _gather_body,
        out_type=jax.ShapeDtypeStruct((_B, _L, _EMB), jnp.float32),
        mesh=mesh,
        scratch_types=(
            [
                pltpu.VMEM((_TOK_W // 2,), jnp.int32),
                pltpu.VMEM((_NBUF, _SPLIT[0], _PAD), jnp.float32),
                pltpu.VMEM((_NBUF, _SPLIT[0], _EMB), jnp.float32),
            ]
            + [pltpu.SemaphoreType.DMA] * (2 * _NBUF)
        ),
        compiler_params=pltpu.CompilerParams(use_tc_tiling_on_sc=True),
    )(tokens_flat, tablep)


def kernel(tokens, table):
    tablep = jnp.pad(table, ((0, 0), (0, _PAD - _EMB)))
    return _embed(tokens.astype(jnp.int32).reshape(-1), tablep)


# speedup vs baseline: 1.0984x; 1.0984x over previous
import math

import jax
import jax.numpy as jnp
from jax import lax
from jax.experimental import pallas as pl
from jax.experimental.pallas import tpu as pltpu
from jax.experimental.pallas import tpu_sc as plsc

_V = 1000000
_EMB = 64
_PAD = 128
_B = 4096
_L = 200
_SCALE = math.sqrt(_EMB)

_NC = 2
_NS = 16
_NW = _NC * _NS

_ROWS_W = _B // _NW
_TOK_W = _ROWS_W * _L
_SPLIT = (104, 96)
_NCHUNK = _ROWS_W
_NBUF = 4
_NOUTER = _NCHUNK // _NBUF


def _gather_body(tokens_hbm, scr_hbm, out_hbm, idx_v, rows, rows_out, *sems):
    sem_g = sems[:_NBUF]
    sem_o = sems[_NBUF:]
    wid = lax.axis_index("s") * _NC + lax.axis_index("c")

    def chunk_geom(b):
        size = _SPLIT[b % 2]
        off = 0 if b % 2 == 0 else _SPLIT[0]
        return size, off

    def idx_chunk(c, b):
        size, off = chunk_geom(b)
        return idx_v.at[pl.ds((c // 2) * _L + off, size)]

    def in_buf(b):
        return rows.at[b, pl.ds(0, chunk_geom(b)[0])]

    def emb_buf(b):
        return rows_out.at[b, pl.ds(0, chunk_geom(b)[0])]

    def run_half(h):
        row0 = wid * _ROWS_W + h * (_ROWS_W // 2)
        pltpu.sync_copy(
            tokens_hbm.at[pl.ds(wid * _TOK_W + h * (_TOK_W // 2), _TOK_W // 2)],
            idx_v,
        )

        def out_chunk(c, b):
            size, off = chunk_geom(b)
            return out_hbm.at[row0 + c // 2, pl.ds(off, size)]

        for b in range(_NBUF):
            pltpu.make_async_copy(
                scr_hbm.at[idx_chunk(b, b)], in_buf(b), sem_g[b]
            ).start()

        def round_body(g, carry):
            for b in range(_NBUF):
                c = g * _NBUF + b
                size, _ = chunk_geom(b)
                pltpu.make_async_copy(
                    scr_hbm.at[idx_chunk(c, b)], in_buf(b), sem_g[b]
                ).wait()

                @pl.when(g > 0)
                def _wait_out():
                    pltpu.make_async_copy(
                        emb_buf(b), out_chunk(c, b), sem_o[b]
                    ).wait()

                def copy_row(r, acc):
                    for j in range(_EMB // 16):
                        rows_out[b, r, pl.ds(j * 16, 16)] = (
                            rows[b, r, pl.ds(j * 16, 16)] * _SCALE
                        )
                    return acc

                lax.fori_loop(0, size, copy_row, 0, unroll=8)

                @pl.when(g < _NOUTER - 1)
                def _next_gather():
                    pltpu.make_async_copy(
                        scr_hbm.at[idx_chunk(c + _NBUF, b)], in_buf(b), sem_g[b]
                    ).start()

                pltpu.make_async_copy(emb_buf(b), out_chunk(c, b), sem_o[b]).start()

            return carry

        lax.fori_loop(0, _NOUTER, round_body, 0)

        for b in range(_NBUF):
            size, off = chunk_geom(b)
            pltpu.make_async_copy(
                emb_buf(b), out_hbm.at[row0, pl.ds(off, size)], sem_o[b]
            ).wait()

    run_half(0)
    run_half(1)


@jax.jit
def _embed(tokens_flat, tablep):
    mesh = plsc.VectorSubcoreMesh(core_axis_name="c", subcore_axis_name="s")
    return pl.kernel(
        _gather_body,
        out_type=jax.ShapeDtypeStruct((_B, _L, _EMB), jnp.float32),
        mesh=mesh,
        scratch_types=(
            [
                pltpu.VMEM((_TOK_W // 2,), jnp.int32),
                pltpu.VMEM((_NBUF, _SPLIT[0], _PAD), jnp.float32),
                pltpu.VMEM((_NBUF, _SPLIT[0], _EMB), jnp.float32),
            ]
            + [pltpu.SemaphoreType.DMA] * (2 * _NBUF)
        ),
        compiler_params=pltpu.CompilerParams(use_tc_tiling_on_sc=True),
    )(tokens_flat, tablep)


def kernel(tokens, table):
    tablep = jnp.pad(table, ((0, 0), (0, _PAD - _EMB)))
    out = _embed(tokens.astype(jnp.int32).reshape(-1), tablep)
    return jax.lax.optimization_barrier(out)
